# baseline (speedup 1.0000x reference)
import functools

import jax
import jax.numpy as jnp
from jax import lax
from jax.experimental import pallas as pl
from jax.experimental.pallas import tpu as pltpu
from jax.experimental.pallas import tpu_sc as plsc

N_PER_GRAPH = 1000
BATCH = 50
N = BATCH * N_PER_GRAPH
E = 800000
F_IN = 16
HID = 64
FH = HID // 2

NC = 2
NS = 16
CHUNK = 128
SUPC = 16
NSUP = 26
ROWS_PER_TILE = NSUP * SUPC
EROWS = ROWS_PER_TILE * NS
EPAD = EROWS * CHUNK
DEG_ROWS_PER_TILE = EROWS // (NC * NS)
DEG_NSUP = DEG_ROWS_PER_TILE // SUPC
INITROWS = 200
NINIT = N // INITROWS
DEGW = 16

_sc_mesh = plsc.VectorSubcoreMesh(core_axis_name="c", subcore_axis_name="s")
_sc_params = pltpu.CompilerParams(use_tc_tiling_on_sc=False)


def _strided_loop(sid, n, body):
    trips = (n - 1) // NS + 1

    def step(t, carry):
        j = sid + t * NS

        @pl.when(j < n)
        def _():
            body(j)

        return carry

    lax.fori_loop(0, trips, step, 0)


def _drain(sid, n, wait_one):
    cnt = (n - 1 - sid) // NS + 1

    def step(t, carry):
        wait_one()
        return carry

    lax.fori_loop(0, cnt, step, 0)


@functools.partial(
    pl.kernel,
    out_type=jax.ShapeDtypeStruct((NC * N, FH), jnp.float32),
    mesh=_sc_mesh,
    scratch_types=[
        pltpu.VMEM((2, SUPC, CHUNK), jnp.int32),
        pltpu.VMEM((2, SUPC, CHUNK), jnp.int32),
        pltpu.VMEM((4, CHUNK, FH), jnp.float32),
        pltpu.VMEM_SHARED((N + 8, FH), jnp.float32),
        pltpu.SemaphoreType.DMA,
        pltpu.SemaphoreType.DMA,
        pltpu.SemaphoreType.DMA,
        pltpu.SemaphoreType.DMA,
        pltpu.SemaphoreType.DMA,
        pltpu.SemaphoreType.DMA,
        pltpu.SemaphoreType.DMA,
        pltpu.SemaphoreType.DMA,
        pltpu.SemaphoreType.DMA,
        pltpu.SemaphoreType.DMA,
        pltpu.SemaphoreType.DMA,
        pltpu.SemaphoreType.DMA,
        pltpu.SemaphoreType.DMA,
    ],
    compiler_params=_sc_params,
)
def _agg_sc(p_hbm, src_hbm, dst_hbm, q_hbm, src_v, dst_v, rows_v, acc_sp,
            ss0, ss1, sd0, sd1, sg0, sg1, sg2, sg3, sc0, sc1, sc2, sc3,
            sinit):
    cid = lax.axis_index("c")
    sid = lax.axis_index("s")
    ssem = (ss0, ss1)
    dsem = (sd0, sd1)
    gsem = (sg0, sg1, sg2, sg3)
    csem = (sc0, sc1, sc2, sc3)

    def init_issue(j):
        r = j * INITROWS
        pltpu.async_copy(p_hbm.at[pl.ds(cid * N + r, INITROWS)],
                         acc_sp.at[pl.ds(r, INITROWS)], sinit)

    _strided_loop(sid, NINIT, init_issue)
    _drain(sid, NINIT,
           lambda: pltpu.make_async_copy(
               p_hbm.at[pl.ds(0, INITROWS)],
               acc_sp.at[pl.ds(0, INITROWS)], sinit).wait())
    plsc.subcore_barrier()

    wrow = sid * ROWS_PER_TILE
    srow = cid * EROWS + wrow

    def issue_idx(s, slot):
        pltpu.async_copy(src_hbm.at[pl.ds(srow + s * SUPC, SUPC)],
                         src_v.at[slot], ssem[slot])
        pltpu.async_copy(dst_hbm.at[pl.ds(wrow + s * SUPC, SUPC)],
                         dst_v.at[slot], dsem[slot])

    def wait_idx(slot):
        pltpu.make_async_copy(src_hbm.at[pl.ds(0, SUPC)], src_v.at[slot],
                              ssem[slot]).wait()
        pltpu.make_async_copy(dst_hbm.at[pl.ds(0, SUPC)], dst_v.at[slot],
                              dsem[slot]).wait()

    def issue_gather(slot, c):
        pltpu.async_copy(p_hbm.at[src_v.at[slot, c]], rows_v.at[c & 3],
                         gsem[c & 3])

    def wait_gather(c):
        pltpu.make_async_copy(p_hbm.at[src_v.at[0, 0]], rows_v.at[c & 3],
                              gsem[c & 3]).wait()

    def issue_scatter(slot, c):
        pltpu.async_copy(rows_v.at[c & 3], acc_sp.at[dst_v.at[slot, c]],
                         csem[c & 3], add=True)

    def wait_scatter(c):
        pltpu.make_async_copy(rows_v.at[c & 3], acc_sp.at[dst_v.at[0, 0]],
                              csem[c & 3]).wait()

    issue_idx(0, 0)

    def do_pair(t, carry):
        for par in (0, 1):
            s = 2 * t + par
            wait_idx(par)

            @pl.when(s + 1 < NSUP)
            def _():
                issue_idx(s + 1, 1 - par)

            for c in range(SUPC):
                if c >= 4:
                    wait_scatter(c)
                else:
                    @pl.when(s > 0)
                    def _():
                        wait_scatter(c)
                issue_gather(par, c)
                if c >= 2:
                    wait_gather(c - 2)
                    issue_scatter(par, c - 2)
            for c in (SUPC - 2, SUPC - 1):
                wait_gather(c)
                issue_scatter(par, c)
        return carry

    lax.fori_loop(0, NSUP // 2, do_pair, 0)
    for c in range(4):
        wait_scatter(c)
    plsc.subcore_barrier()

    def wb_issue(j):
        r = j * INITROWS
        pltpu.async_copy(acc_sp.at[pl.ds(r, INITROWS)],
                         q_hbm.at[pl.ds(cid * N + r, INITROWS)], sinit)

    _strided_loop(sid, NINIT, wb_issue)
    _drain(sid, NINIT,
           lambda: pltpu.make_async_copy(
               acc_sp.at[pl.ds(0, INITROWS)],
               q_hbm.at[pl.ds(0, INITROWS)], sinit).wait())


ROWS_TC = 5000
NB = N // ROWS_TC


def _dinv_of(deg_ref):
    return lax.rsqrt(deg_ref[:, 0:1])


def _leaky(t):
    return jnp.where(t > 0, t, 0.2 * t)


def _half_sel(h, c):
    return jnp.where(c == 0, h[:, :FH], h[:, FH:])


def _tc_first_body(x_ref, deg_ref, w_ref, p_ref):
    c = pl.program_id(0)
    dinv = _dinv_of(deg_ref)
    h = jnp.dot(x_ref[...], w_ref[...], preferred_element_type=jnp.float32)
    p_ref[...] = _half_sel(h, c) * dinv


_tc_first = pl.pallas_call(
    _tc_first_body,
    grid=(NC, NB),
    in_specs=[
        pl.BlockSpec((ROWS_TC, F_IN), lambda c, i: (i, 0)),
        pl.BlockSpec((ROWS_TC, FH), lambda c, i: (i, 0)),
        pl.BlockSpec((F_IN, HID), lambda c, i: (0, 0)),
    ],
    out_specs=pl.BlockSpec((ROWS_TC, FH), lambda c, i: (c * NB + i, 0)),
    out_shape=jax.ShapeDtypeStruct((NC * N, FH), jnp.float32),
)


def _tc_mid_body(qa_ref, qb_ref, deg_ref, b_ref, w_ref, p_ref):
    c = pl.program_id(0)
    dinv = _dinv_of(deg_ref)
    act0 = _leaky(qa_ref[...] * dinv + b_ref[0:1, :])
    act1 = _leaky(qb_ref[...] * dinv + b_ref[1:2, :])
    act = jnp.concatenate([act0, act1], axis=1)
    h = jnp.dot(act, w_ref[...], preferred_element_type=jnp.float32)
    p_ref[...] = _half_sel(h, c) * dinv


_tc_mid = pl.pallas_call(
    _tc_mid_body,
    grid=(NC, NB),
    in_specs=[
        pl.BlockSpec((ROWS_TC, FH), lambda c, i: (i, 0)),
        pl.BlockSpec((ROWS_TC, FH), lambda c, i: (i + NB, 0)),
        pl.BlockSpec((ROWS_TC, FH), lambda c, i: (i, 0)),
        pl.BlockSpec((NC, FH), lambda c, i: (0, 0)),
        pl.BlockSpec((HID, HID), lambda c, i: (0, 0)),
    ],
    out_specs=pl.BlockSpec((ROWS_TC, FH), lambda c, i: (c * NB + i, 0)),
    out_shape=jax.ShapeDtypeStruct((NC * N, FH), jnp.float32),
)


def _tc_final_body(qa_ref, qb_ref, deg_ref, b_ref, wfca_ref, wfcb_ref,
                   bfc_ref, out_ref):
    dinv = _dinv_of(deg_ref)
    act0 = _leaky(qa_ref[...] * dinv + b_ref[0:1, :])
    act1 = _leaky(qb_ref[...] * dinv + b_ref[1:2, :])
    s = (jnp.sum(act0 * wfca_ref[...]) + jnp.sum(act1 * wfcb_ref[...])
         + bfc_ref[0, 0])
    out_ref[...] = jnp.broadcast_to(jax.nn.sigmoid(s), (1, 1, 128))


GB = N_PER_GRAPH
NGB = N // GB

_tc_final = pl.pallas_call(
    _tc_final_body,
    grid=(BATCH,),
    in_specs=[
        pl.BlockSpec((GB, FH), lambda g: (g, 0)),
        pl.BlockSpec((GB, FH), lambda g: (g + NGB, 0)),
        pl.BlockSpec((GB, FH), lambda g: (g, 0)),
        pl.BlockSpec((NC, FH), lambda g: (0, 0)),
        pl.BlockSpec((GB, FH), lambda g: (0, 0)),
        pl.BlockSpec((GB, FH), lambda g: (1, 0)),
        pl.BlockSpec((1, 128), lambda g: (0, 0)),
    ],
    out_specs=pl.BlockSpec((1, 1, 128), lambda g: (g, 0, 0)),
    out_shape=jax.ShapeDtypeStruct((BATCH, 1, 128), jnp.float32),
)


def kernel(x, edge_list, W1, b1, W2, b2, W3, b3, Wfc, bfc):
    npad = EPAD - E
    src2d = jnp.concatenate(
        [edge_list[0], jnp.zeros((npad,), jnp.int32)]).reshape(EROWS, CHUNK)
    dst2d = jnp.concatenate(
        [edge_list[1], jnp.full((npad,), N, jnp.int32)]).reshape(EROWS, CHUNK)
    src_both = jnp.concatenate([src2d, src2d + N], axis=0)
    b1s = b1.reshape(NC, FH)
    b2s = b2.reshape(NC, FH)
    b3s = b3.reshape(NC, FH)
    wfc2 = Wfc.reshape(GB, NC, FH).transpose(1, 0, 2).reshape(NC * GB, FH)
    bfcr = jnp.broadcast_to(bfc.reshape(1, 1), (1, 128))
    ones_p = jnp.ones((NC * N, FH), jnp.float32)

    qdeg = _agg_sc(ones_p, src_both, dst2d)
    p1 = _tc_first(x, qdeg, W1)
    q1 = _agg_sc(p1, src_both, dst2d)
    p2 = _tc_mid(q1, q1, qdeg, b1s, W2)
    q2 = _agg_sc(p2, src_both, dst2d)
    p3 = _tc_mid(q2, q2, qdeg, b2s, W3)
    q3 = _agg_sc(p3, src_both, dst2d)
    out = _tc_final(q3, q3, qdeg, b3s, wfc2, wfc2, bfcr)
    return out[:, 0, 0]


# speedup vs baseline: 1.1007x; 1.1007x over previous
import functools

import jax
import jax.numpy as jnp
from jax import lax
from jax.experimental import pallas as pl
from jax.experimental.pallas import tpu as pltpu
from jax.experimental.pallas import tpu_sc as plsc

N_PER_GRAPH = 1000
BATCH = 50
N = BATCH * N_PER_GRAPH
E = 800000
F_IN = 16
HID = 64
FH = HID // 2

NC = 2
NS = 16
CHUNK = 128
SUPC = 16
NSUP = 26
ROWS_PER_TILE = NSUP * SUPC
EROWS = ROWS_PER_TILE * NS
EPAD = EROWS * CHUNK
DEG_ROWS_PER_TILE = EROWS // (NC * NS)
DEG_NSUP = DEG_ROWS_PER_TILE // SUPC
INITROWS = 200
NINIT = N // INITROWS
DEGW = 16

_sc_mesh = plsc.VectorSubcoreMesh(core_axis_name="c", subcore_axis_name="s")
_sc_params = pltpu.CompilerParams(use_tc_tiling_on_sc=False)


def _strided_loop(sid, n, body):
    trips = (n - 1) // NS + 1

    def step(t, carry):
        j = sid + t * NS

        @pl.when(j < n)
        def _():
            body(j)

        return carry

    lax.fori_loop(0, trips, step, 0)


def _drain(sid, n, wait_one):
    cnt = (n - 1 - sid) // NS + 1

    def step(t, carry):
        wait_one()
        return carry

    lax.fori_loop(0, cnt, step, 0)


@functools.partial(
    pl.kernel,
    out_type=jax.ShapeDtypeStruct((NC * N, DEGW), jnp.float32),
    mesh=_sc_mesh,
    scratch_types=[
        pltpu.VMEM((2, SUPC, CHUNK), jnp.int32),
        pltpu.VMEM((CHUNK, DEGW), jnp.float32),
        pltpu.VMEM((INITROWS, DEGW), jnp.float32),
        pltpu.VMEM_SHARED((N + 8, DEGW), jnp.float32),
        pltpu.SemaphoreType.DMA,
        pltpu.SemaphoreType.DMA,
        pltpu.SemaphoreType.DMA,
        pltpu.SemaphoreType.DMA,
    ],
    compiler_params=_sc_params,
)
def _deg_sc(dst_hbm, deg_hbm, dst_v, ones_v, zeros_v, acc_sp, sd0, sd1, ssc,
            sinit):
    cid = lax.axis_index("c")
    sid = lax.axis_index("s")
    dsem = (sd0, sd1)

    def fill(i, _):
        ones_v[i] = jnp.full((DEGW,), 1.0, jnp.float32)
        return _

    lax.fori_loop(0, CHUNK, fill, 0)

    def fillz(i, _):
        zeros_v[i] = jnp.zeros((DEGW,), jnp.float32)
        return _

    lax.fori_loop(0, INITROWS, fillz, 0)

    def zero_chunk(j):
        pltpu.async_copy(zeros_v, acc_sp.at[pl.ds(j * INITROWS, INITROWS)],
                         sinit)

    _strided_loop(sid, NINIT, zero_chunk)
    _drain(sid, NINIT,
           lambda: pltpu.make_async_copy(
               zeros_v, acc_sp.at[pl.ds(0, INITROWS)], sinit).wait())
    plsc.subcore_barrier()

    wrow = (cid * NS + sid) * DEG_ROWS_PER_TILE

    def issue_idx(s, slot):
        pltpu.async_copy(dst_hbm.at[pl.ds(wrow + s * SUPC, SUPC)],
                         dst_v.at[slot], dsem[slot])

    issue_idx(0, 0)

    def do_pair(t, carry):
        for par in (0, 1):
            s = 2 * t + par

            @pl.when(s < DEG_NSUP)
            def _():
                pltpu.make_async_copy(dst_hbm.at[pl.ds(0, SUPC)],
                                      dst_v.at[par], dsem[par]).wait()

                @pl.when(s + 1 < DEG_NSUP)
                def _():
                    issue_idx(s + 1, 1 - par)

                descs = []
                for c in range(SUPC):
                    descs.append(
                        pltpu.async_copy(ones_v, acc_sp.at[dst_v.at[par, c]],
                                         ssc, add=True))
                for d in descs:
                    d.wait()
        return carry

    lax.fori_loop(0, (DEG_NSUP + 1) // 2, do_pair, 0)
    plsc.subcore_barrier()

    def writeback(j):
        r = j * INITROWS
        pltpu.async_copy(acc_sp.at[pl.ds(r, INITROWS)],
                         deg_hbm.at[pl.ds(cid * N + r, INITROWS)], sinit)

    _strided_loop(sid, NINIT, writeback)
    _drain(sid, NINIT,
           lambda: pltpu.make_async_copy(
               acc_sp.at[pl.ds(0, INITROWS)],
               deg_hbm.at[pl.ds(0, INITROWS)], sinit).wait())


@functools.partial(
    pl.kernel,
    out_type=jax.ShapeDtypeStruct((NC * N, FH), jnp.float32),
    mesh=_sc_mesh,
    scratch_types=[
        pltpu.VMEM((2, SUPC, CHUNK), jnp.int32),
        pltpu.VMEM((2, SUPC, CHUNK), jnp.int32),
        pltpu.VMEM((4, CHUNK, FH), jnp.float32),
        pltpu.VMEM_SHARED((N + 8, FH), jnp.float32),
        pltpu.SemaphoreType.DMA,
        pltpu.SemaphoreType.DMA,
        pltpu.SemaphoreType.DMA,
        pltpu.SemaphoreType.DMA,
        pltpu.SemaphoreType.DMA,
        pltpu.SemaphoreType.DMA,
        pltpu.SemaphoreType.DMA,
        pltpu.SemaphoreType.DMA,
        pltpu.SemaphoreType.DMA,
        pltpu.SemaphoreType.DMA,
        pltpu.SemaphoreType.DMA,
        pltpu.SemaphoreType.DMA,
        pltpu.SemaphoreType.DMA,
    ],
    compiler_params=_sc_params,
)
def _agg_sc(p_hbm, src_hbm, dst_hbm, q_hbm, src_v, dst_v, rows_v, acc_sp,
            ss0, ss1, sd0, sd1, sg0, sg1, sg2, sg3, sc0, sc1, sc2, sc3,
            sinit):
    cid = lax.axis_index("c")
    sid = lax.axis_index("s")
    ssem = (ss0, ss1)
    dsem = (sd0, sd1)
    gsem = (sg0, sg1, sg2, sg3)
    csem = (sc0, sc1, sc2, sc3)

    def init_issue(j):
        r = j * INITROWS
        pltpu.async_copy(p_hbm.at[pl.ds(cid * N + r, INITROWS)],
                         acc_sp.at[pl.ds(r, INITROWS)], sinit)

    _strided_loop(sid, NINIT, init_issue)
    _drain(sid, NINIT,
           lambda: pltpu.make_async_copy(
               p_hbm.at[pl.ds(0, INITROWS)],
               acc_sp.at[pl.ds(0, INITROWS)], sinit).wait())
    plsc.subcore_barrier()

    wrow = sid * ROWS_PER_TILE
    srow = cid * EROWS + wrow

    def issue_idx(s, slot):
        pltpu.async_copy(src_hbm.at[pl.ds(srow + s * SUPC, SUPC)],
                         src_v.at[slot], ssem[slot])
        pltpu.async_copy(dst_hbm.at[pl.ds(wrow + s * SUPC, SUPC)],
                         dst_v.at[slot], dsem[slot])

    def wait_idx(slot):
        pltpu.make_async_copy(src_hbm.at[pl.ds(0, SUPC)], src_v.at[slot],
                              ssem[slot]).wait()
        pltpu.make_async_copy(dst_hbm.at[pl.ds(0, SUPC)], dst_v.at[slot],
                              dsem[slot]).wait()

    def issue_gather(slot, c):
        pltpu.async_copy(p_hbm.at[src_v.at[slot, c]], rows_v.at[c & 3],
                         gsem[c & 3])

    def wait_gather(c):
        pltpu.make_async_copy(p_hbm.at[src_v.at[0, 0]], rows_v.at[c & 3],
                              gsem[c & 3]).wait()

    def issue_scatter(slot, c):
        pltpu.async_copy(rows_v.at[c & 3], acc_sp.at[dst_v.at[slot, c]],
                         csem[c & 3], add=True)

    def wait_scatter(c):
        pltpu.make_async_copy(rows_v.at[c & 3], acc_sp.at[dst_v.at[0, 0]],
                              csem[c & 3]).wait()

    issue_idx(0, 0)

    def do_pair(t, carry):
        for par in (0, 1):
            s = 2 * t + par
            wait_idx(par)

            @pl.when(s + 1 < NSUP)
            def _():
                issue_idx(s + 1, 1 - par)

            for c in range(SUPC):
                if c >= 4:
                    wait_scatter(c)
                else:
                    @pl.when(s > 0)
                    def _():
                        wait_scatter(c)
                issue_gather(par, c)
                if c >= 2:
                    wait_gather(c - 2)
                    issue_scatter(par, c - 2)
            for c in (SUPC - 2, SUPC - 1):
                wait_gather(c)
                issue_scatter(par, c)
        return carry

    lax.fori_loop(0, NSUP // 2, do_pair, 0)
    for c in range(4):
        wait_scatter(c)
    plsc.subcore_barrier()

    def wb_issue(j):
        r = j * INITROWS
        pltpu.async_copy(acc_sp.at[pl.ds(r, INITROWS)],
                         q_hbm.at[pl.ds(cid * N + r, INITROWS)], sinit)

    _strided_loop(sid, NINIT, wb_issue)
    _drain(sid, NINIT,
           lambda: pltpu.make_async_copy(
               acc_sp.at[pl.ds(0, INITROWS)],
               q_hbm.at[pl.ds(0, INITROWS)], sinit).wait())


ROWS_TC = 5000
NB = N // ROWS_TC


def _dinv_of(dega_ref, degb_ref):
    return lax.rsqrt(dega_ref[:, 0:1] + degb_ref[:, 0:1] + 1.0)


def _leaky(t):
    return jnp.where(t > 0, t, 0.2 * t)


def _half_sel(h, c):
    return jnp.where(c == 0, h[:, :FH], h[:, FH:])


def _tc_first_body(x_ref, dega_ref, degb_ref, w_ref, p_ref):
    c = pl.program_id(0)
    dinv = _dinv_of(dega_ref, degb_ref)
    h = jnp.dot(x_ref[...], w_ref[...], preferred_element_type=jnp.float32)
    p_ref[...] = _half_sel(h, c) * dinv


_tc_first = pl.pallas_call(
    _tc_first_body,
    grid=(NC, NB),
    in_specs=[
        pl.BlockSpec((ROWS_TC, F_IN), lambda c, i: (i, 0)),
        pl.BlockSpec((ROWS_TC, DEGW), lambda c, i: (i, 0)),
        pl.BlockSpec((ROWS_TC, DEGW), lambda c, i: (i + NB, 0)),
        pl.BlockSpec((F_IN, HID), lambda c, i: (0, 0)),
    ],
    out_specs=pl.BlockSpec((ROWS_TC, FH), lambda c, i: (c * NB + i, 0)),
    out_shape=jax.ShapeDtypeStruct((NC * N, FH), jnp.float32),
)


def _tc_mid_body(qa_ref, qb_ref, dega_ref, degb_ref, b_ref, w_ref, p_ref):
    c = pl.program_id(0)
    dinv = _dinv_of(dega_ref, degb_ref)
    act0 = _leaky(qa_ref[...] * dinv + b_ref[0:1, :])
    act1 = _leaky(qb_ref[...] * dinv + b_ref[1:2, :])
    act = jnp.concatenate([act0, act1], axis=1)
    h = jnp.dot(act, w_ref[...], preferred_element_type=jnp.float32)
    p_ref[...] = _half_sel(h, c) * dinv


_tc_mid = pl.pallas_call(
    _tc_mid_body,
    grid=(NC, NB),
    in_specs=[
        pl.BlockSpec((ROWS_TC, FH), lambda c, i: (i, 0)),
        pl.BlockSpec((ROWS_TC, FH), lambda c, i: (i + NB, 0)),
        pl.BlockSpec((ROWS_TC, DEGW), lambda c, i: (i, 0)),
        pl.BlockSpec((ROWS_TC, DEGW), lambda c, i: (i + NB, 0)),
        pl.BlockSpec((NC, FH), lambda c, i: (0, 0)),
        pl.BlockSpec((HID, HID), lambda c, i: (0, 0)),
    ],
    out_specs=pl.BlockSpec((ROWS_TC, FH), lambda c, i: (c * NB + i, 0)),
    out_shape=jax.ShapeDtypeStruct((NC * N, FH), jnp.float32),
)


def _tc_final_body(qa_ref, qb_ref, dega_ref, degb_ref, b_ref, wfca_ref,
                   wfcb_ref, bfc_ref, out_ref):
    dinv = _dinv_of(dega_ref, degb_ref)
    act0 = _leaky(qa_ref[...] * dinv + b_ref[0:1, :])
    act1 = _leaky(qb_ref[...] * dinv + b_ref[1:2, :])
    s = (jnp.sum(act0 * wfca_ref[...]) + jnp.sum(act1 * wfcb_ref[...])
         + bfc_ref[0, 0])
    out_ref[...] = jnp.broadcast_to(jax.nn.sigmoid(s), (1, 1, 128))


GB = N_PER_GRAPH
NGB = N // GB

_tc_final = pl.pallas_call(
    _tc_final_body,
    grid=(BATCH,),
    in_specs=[
        pl.BlockSpec((GB, FH), lambda g: (g, 0)),
        pl.BlockSpec((GB, FH), lambda g: (g + NGB, 0)),
        pl.BlockSpec((GB, DEGW), lambda g: (g, 0)),
        pl.BlockSpec((GB, DEGW), lambda g: (g + NGB, 0)),
        pl.BlockSpec((NC, FH), lambda g: (0, 0)),
        pl.BlockSpec((GB, FH), lambda g: (0, 0)),
        pl.BlockSpec((GB, FH), lambda g: (1, 0)),
        pl.BlockSpec((1, 128), lambda g: (0, 0)),
    ],
    out_specs=pl.BlockSpec((1, 1, 128), lambda g: (g, 0, 0)),
    out_shape=jax.ShapeDtypeStruct((BATCH, 1, 128), jnp.float32),
)


def kernel(x, edge_list, W1, b1, W2, b2, W3, b3, Wfc, bfc):
    npad = EPAD - E
    src2d = jnp.concatenate(
        [edge_list[0], jnp.zeros((npad,), jnp.int32)]).reshape(EROWS, CHUNK)
    dst2d = jnp.concatenate(
        [edge_list[1], jnp.full((npad,), N, jnp.int32)]).reshape(EROWS, CHUNK)
    src_both = jnp.concatenate([src2d, src2d + N], axis=0)
    b1s = b1.reshape(NC, FH)
    b2s = b2.reshape(NC, FH)
    b3s = b3.reshape(NC, FH)
    wfc2 = Wfc.reshape(GB, NC, FH).transpose(1, 0, 2).reshape(NC * GB, FH)
    bfcr = jnp.broadcast_to(bfc.reshape(1, 1), (1, 128))
    deg2 = _deg_sc(dst2d)
    p1 = _tc_first(x, deg2, deg2, W1)
    q1 = _agg_sc(p1, src_both, dst2d)
    p2 = _tc_mid(q1, q1, deg2, deg2, b1s, W2)
    q2 = _agg_sc(p2, src_both, dst2d)
    p3 = _tc_mid(q2, q2, deg2, deg2, b2s, W3)
    q3 = _agg_sc(p3, src_both, dst2d)
    out = _tc_final(q3, q3, deg2, deg2, b3s, wfc2, wfc2, bfcr)
    return out[:, 0, 0]


# speedup vs baseline: 1.1186x; 1.0162x over previous
import functools

import jax
import jax.numpy as jnp
from jax import lax
from jax.experimental import pallas as pl
from jax.experimental.pallas import tpu as pltpu
from jax.experimental.pallas import tpu_sc as plsc

N_PER_GRAPH = 1000
BATCH = 50
N = BATCH * N_PER_GRAPH
E = 800000
F_IN = 16
HID = 64
FH = HID // 2

NC = 2
NS = 16
CHUNK = 128
SUPC = 16
NSUP = 26
ROWS_PER_TILE = NSUP * SUPC
EROWS = ROWS_PER_TILE * NS
EPAD = EROWS * CHUNK
DEG_ROWS_PER_TILE = EROWS // (NC * NS)
DEG_NSUP = DEG_ROWS_PER_TILE // SUPC
INITROWS = 200
NINIT = N // INITROWS
DEGW = 16

_sc_mesh = plsc.VectorSubcoreMesh(core_axis_name="c", subcore_axis_name="s")
_sc_params = pltpu.CompilerParams(use_tc_tiling_on_sc=False)


def _strided_loop(sid, n, body):
    trips = (n - 1) // NS + 1

    def step(t, carry):
        j = sid + t * NS

        @pl.when(j < n)
        def _():
            body(j)

        return carry

    lax.fori_loop(0, trips, step, 0)


def _drain(sid, n, wait_one):
    cnt = (n - 1 - sid) // NS + 1

    def step(t, carry):
        wait_one()
        return carry

    lax.fori_loop(0, cnt, step, 0)


@functools.partial(
    pl.kernel,
    out_type=jax.ShapeDtypeStruct((NC * N, DEGW), jnp.float32),
    mesh=_sc_mesh,
    scratch_types=[
        pltpu.VMEM((2, SUPC, CHUNK), jnp.int32),
        pltpu.VMEM((CHUNK, DEGW), jnp.float32),
        pltpu.VMEM((INITROWS, DEGW), jnp.float32),
        pltpu.VMEM_SHARED((N + 8, DEGW), jnp.float32),
        pltpu.SemaphoreType.DMA,
        pltpu.SemaphoreType.DMA,
        pltpu.SemaphoreType.DMA,
        pltpu.SemaphoreType.DMA,
    ],
    compiler_params=_sc_params,
)
def _deg_sc(dst_hbm, deg_hbm, dst_v, ones_v, zeros_v, acc_sp, sd0, sd1, ssc,
            sinit):
    cid = lax.axis_index("c")
    sid = lax.axis_index("s")
    dsem = (sd0, sd1)

    def fill(i, _):
        ones_v[i] = jnp.full((DEGW,), 1.0, jnp.float32)
        return _

    lax.fori_loop(0, CHUNK, fill, 0)

    def fillz(i, _):
        zeros_v[i] = jnp.zeros((DEGW,), jnp.float32)
        return _

    lax.fori_loop(0, INITROWS, fillz, 0)

    def zero_chunk(j):
        pltpu.async_copy(zeros_v, acc_sp.at[pl.ds(j * INITROWS, INITROWS)],
                         sinit)

    _strided_loop(sid, NINIT, zero_chunk)
    _drain(sid, NINIT,
           lambda: pltpu.make_async_copy(
               zeros_v, acc_sp.at[pl.ds(0, INITROWS)], sinit).wait())
    plsc.subcore_barrier()

    wrow = (cid * NS + sid) * DEG_ROWS_PER_TILE

    def issue_idx(s, slot):
        pltpu.async_copy(dst_hbm.at[pl.ds(wrow + s * SUPC, SUPC)],
                         dst_v.at[slot], dsem[slot])

    issue_idx(0, 0)

    def do_pair(t, carry):
        for par in (0, 1):
            s = 2 * t + par

            @pl.when(s < DEG_NSUP)
            def _():
                pltpu.make_async_copy(dst_hbm.at[pl.ds(0, SUPC)],
                                      dst_v.at[par], dsem[par]).wait()

                @pl.when(s + 1 < DEG_NSUP)
                def _():
                    issue_idx(s + 1, 1 - par)

                descs = []
                for c in range(SUPC):
                    descs.append(
                        pltpu.async_copy(ones_v, acc_sp.at[dst_v.at[par, c]],
                                         ssc, add=True))
                for d in descs:
                    d.wait()
        return carry

    lax.fori_loop(0, (DEG_NSUP + 1) // 2, do_pair, 0)
    plsc.subcore_barrier()

    def writeback(j):
        r = j * INITROWS
        pltpu.async_copy(acc_sp.at[pl.ds(r, INITROWS)],
                         deg_hbm.at[pl.ds(cid * N + r, INITROWS)], sinit)

    _strided_loop(sid, NINIT, writeback)
    _drain(sid, NINIT,
           lambda: pltpu.make_async_copy(
               acc_sp.at[pl.ds(0, INITROWS)],
               deg_hbm.at[pl.ds(0, INITROWS)], sinit).wait())


@functools.partial(
    pl.kernel,
    out_type=jax.ShapeDtypeStruct((NC * N, FH), jnp.float32),
    mesh=_sc_mesh,
    scratch_types=[
        pltpu.VMEM((2, SUPC, CHUNK), jnp.int32),
        pltpu.VMEM((2, SUPC, CHUNK), jnp.int32),
        pltpu.VMEM((4, CHUNK, FH), jnp.float32),
        pltpu.VMEM_SHARED((N + 8, FH), jnp.float32),
        pltpu.SemaphoreType.DMA,
        pltpu.SemaphoreType.DMA,
        pltpu.SemaphoreType.DMA,
        pltpu.SemaphoreType.DMA,
        pltpu.SemaphoreType.DMA,
        pltpu.SemaphoreType.DMA,
        pltpu.SemaphoreType.DMA,
        pltpu.SemaphoreType.DMA,
        pltpu.SemaphoreType.DMA,
        pltpu.SemaphoreType.DMA,
        pltpu.SemaphoreType.DMA,
        pltpu.SemaphoreType.DMA,
        pltpu.SemaphoreType.DMA,
    ],
    compiler_params=_sc_params,
)
def _agg_sc(p_hbm, src_hbm, dst_hbm, q_hbm, src_v, dst_v, rows_v, acc_sp,
            ss0, ss1, sd0, sd1, sg0, sg1, sg2, sg3, sc0, sc1, sc2, sc3,
            sinit):
    cid = lax.axis_index("c")
    sid = lax.axis_index("s")
    ssem = (ss0, ss1)
    dsem = (sd0, sd1)
    gsem = (sg0, sg1, sg2, sg3)
    csem = (sc0, sc1, sc2, sc3)

    def init_issue(j):
        r = j * INITROWS
        pltpu.async_copy(p_hbm.at[pl.ds(cid * N + r, INITROWS)],
                         acc_sp.at[pl.ds(r, INITROWS)], sinit)

    _strided_loop(sid, NINIT, init_issue)
    _drain(sid, NINIT,
           lambda: pltpu.make_async_copy(
               p_hbm.at[pl.ds(0, INITROWS)],
               acc_sp.at[pl.ds(0, INITROWS)], sinit).wait())
    plsc.subcore_barrier()

    wrow = sid * ROWS_PER_TILE
    srow = cid * EROWS + wrow

    def issue_idx(s, slot):
        pltpu.async_copy(src_hbm.at[pl.ds(srow + s * SUPC, SUPC)],
                         src_v.at[slot], ssem[slot])
        pltpu.async_copy(dst_hbm.at[pl.ds(wrow + s * SUPC, SUPC)],
                         dst_v.at[slot], dsem[slot])

    def wait_idx(slot):
        pltpu.make_async_copy(src_hbm.at[pl.ds(0, SUPC)], src_v.at[slot],
                              ssem[slot]).wait()
        pltpu.make_async_copy(dst_hbm.at[pl.ds(0, SUPC)], dst_v.at[slot],
                              dsem[slot]).wait()

    def issue_gather(slot, c):
        pltpu.async_copy(p_hbm.at[src_v.at[slot, c]], rows_v.at[c & 3],
                         gsem[c & 3])

    def wait_gather(c):
        pltpu.make_async_copy(p_hbm.at[src_v.at[0, 0]], rows_v.at[c & 3],
                              gsem[c & 3]).wait()

    def issue_scatter(slot, c):
        pltpu.async_copy(rows_v.at[c & 3], acc_sp.at[dst_v.at[slot, c]],
                         csem[c & 3], add=True)

    def wait_scatter(c):
        pltpu.make_async_copy(rows_v.at[c & 3], acc_sp.at[dst_v.at[0, 0]],
                              csem[c & 3]).wait()

    issue_idx(0, 0)

    def do_pair(t, carry):
        for par in (0, 1):
            s = 2 * t + par
            wait_idx(par)

            @pl.when(s + 1 < NSUP)
            def _():
                issue_idx(s + 1, 1 - par)

            for c in range(SUPC):
                if c >= 4:
                    wait_scatter(c)
                else:
                    @pl.when(s > 0)
                    def _():
                        wait_scatter(c)
                issue_gather(par, c)
                if c >= 2:
                    wait_gather(c - 2)
                    issue_scatter(par, c - 2)
            for c in (SUPC - 2, SUPC - 1):
                wait_gather(c)
                issue_scatter(par, c)
        return carry

    lax.fori_loop(0, NSUP // 2, do_pair, 0)
    for c in range(4):
        wait_scatter(c)
    plsc.subcore_barrier()

    def wb_issue(j):
        r = j * INITROWS
        pltpu.async_copy(acc_sp.at[pl.ds(r, INITROWS)],
                         q_hbm.at[pl.ds(cid * N + r, INITROWS)], sinit)

    _strided_loop(sid, NINIT, wb_issue)
    _drain(sid, NINIT,
           lambda: pltpu.make_async_copy(
               acc_sp.at[pl.ds(0, INITROWS)],
               q_hbm.at[pl.ds(0, INITROWS)], sinit).wait())


ROWS_TC = 5000
NB = N // ROWS_TC


def _dinv_of(dega_ref, degb_ref):
    return lax.rsqrt(dega_ref[:, 0:1] + degb_ref[:, 0:1] + 1.0)


def _leaky(t):
    return jnp.where(t > 0, t, 0.2 * t)


def _half_sel(h, c):
    return jnp.where(c == 0, h[:, :FH], h[:, FH:])


def _tc_first_body(x_ref, dega_ref, degb_ref, w_ref, p_ref):
    c = pl.program_id(0)
    dinv = _dinv_of(dega_ref, degb_ref)
    h = jnp.dot(x_ref[...], w_ref[...], preferred_element_type=jnp.float32)
    p_ref[...] = _half_sel(h, c) * dinv


_tc_first = pl.pallas_call(
    _tc_first_body,
    grid=(NC, NB),
    in_specs=[
        pl.BlockSpec((ROWS_TC, F_IN), lambda c, i: (i, 0)),
        pl.BlockSpec((ROWS_TC, DEGW), lambda c, i: (i, 0)),
        pl.BlockSpec((ROWS_TC, DEGW), lambda c, i: (i + NB, 0)),
        pl.BlockSpec((F_IN, HID), lambda c, i: (0, 0)),
    ],
    out_specs=pl.BlockSpec((ROWS_TC, FH), lambda c, i: (c * NB + i, 0)),
    out_shape=jax.ShapeDtypeStruct((NC * N, FH), jnp.float32),
)


def _tc_mid_body(qa_ref, qb_ref, dega_ref, degb_ref, b_ref, w_ref, p_ref):
    c = pl.program_id(0)
    dinv = _dinv_of(dega_ref, degb_ref)
    act0 = _leaky(qa_ref[...] * dinv + b_ref[0:1, :])
    act1 = _leaky(qb_ref[...] * dinv + b_ref[1:2, :])
    act = jnp.concatenate([act0, act1], axis=1)
    h = jnp.dot(act, w_ref[...], preferred_element_type=jnp.float32)
    p_ref[...] = _half_sel(h, c) * dinv


_tc_mid = pl.pallas_call(
    _tc_mid_body,
    grid=(NC, NB),
    in_specs=[
        pl.BlockSpec((ROWS_TC, FH), lambda c, i: (i, 0)),
        pl.BlockSpec((ROWS_TC, FH), lambda c, i: (i + NB, 0)),
        pl.BlockSpec((ROWS_TC, DEGW), lambda c, i: (i, 0)),
        pl.BlockSpec((ROWS_TC, DEGW), lambda c, i: (i + NB, 0)),
        pl.BlockSpec((NC, FH), lambda c, i: (0, 0)),
        pl.BlockSpec((HID, HID), lambda c, i: (0, 0)),
    ],
    out_specs=pl.BlockSpec((ROWS_TC, FH), lambda c, i: (c * NB + i, 0)),
    out_shape=jax.ShapeDtypeStruct((NC * N, FH), jnp.float32),
)


def _tc_final_body(qa_ref, qb_ref, dega_ref, degb_ref, b_ref, wfca_ref,
                   wfcb_ref, bfc_ref, out_ref):
    dinv = _dinv_of(dega_ref, degb_ref)
    act0 = _leaky(qa_ref[...] * dinv + b_ref[0:1, :])
    act1 = _leaky(qb_ref[...] * dinv + b_ref[1:2, :])
    s = (jnp.sum(act0 * wfca_ref[...]) + jnp.sum(act1 * wfcb_ref[...])
         + bfc_ref[0, 0])
    out_ref[...] = jnp.broadcast_to(jax.nn.sigmoid(s), (1, 1, 128))


GB = N_PER_GRAPH
NGB = N // GB

_tc_final = pl.pallas_call(
    _tc_final_body,
    grid=(BATCH,),
    in_specs=[
        pl.BlockSpec((GB, FH), lambda g: (g, 0)),
        pl.BlockSpec((GB, FH), lambda g: (g + NGB, 0)),
        pl.BlockSpec((GB, DEGW), lambda g: (g, 0)),
        pl.BlockSpec((GB, DEGW), lambda g: (g + NGB, 0)),
        pl.BlockSpec((NC, FH), lambda g: (0, 0)),
        pl.BlockSpec((GB, FH), lambda g: (0, 0)),
        pl.BlockSpec((GB, FH), lambda g: (1, 0)),
        pl.BlockSpec((1, 128), lambda g: (0, 0)),
    ],
    out_specs=pl.BlockSpec((1, 1, 128), lambda g: (g, 0, 0)),
    out_shape=jax.ShapeDtypeStruct((BATCH, 1, 128), jnp.float32),
)


def kernel(x, edge_list, W1, b1, W2, b2, W3, b3, Wfc, bfc):
    npad = EPAD - E
    src2d = jnp.concatenate(
        [edge_list[0], jnp.zeros((npad,), jnp.int32)]).reshape(EROWS, CHUNK)
    padv = N + (jnp.arange(npad, dtype=jnp.int32) & 7)
    dst2d = jnp.concatenate(
        [edge_list[1], padv]).reshape(EROWS, CHUNK)
    src_both = jnp.concatenate([src2d, src2d + N], axis=0)
    b1s = b1.reshape(NC, FH)
    b2s = b2.reshape(NC, FH)
    b3s = b3.reshape(NC, FH)
    wfc2 = Wfc.reshape(GB, NC, FH).transpose(1, 0, 2).reshape(NC * GB, FH)
    bfcr = jnp.broadcast_to(bfc.reshape(1, 1), (1, 128))
    deg2 = _deg_sc(dst2d)
    p1 = _tc_first(x, deg2, deg2, W1)
    q1 = _agg_sc(p1, src_both, dst2d)
    p2 = _tc_mid(q1, q1, deg2, deg2, b1s, W2)
    q2 = _agg_sc(p2, src_both, dst2d)
    p3 = _tc_mid(q2, q2, deg2, deg2, b2s, W3)
    q3 = _agg_sc(p3, src_both, dst2d)
    out = _tc_final(q3, q3, deg2, deg2, b3s, wfc2, wfc2, bfcr)
    return out[:, 0, 0]
